# Initial kernel scaffold; baseline (speedup 1.0000x reference)
import jax
import jax.numpy as jnp
from jax.experimental import pallas as pl


def kernel(x, depth, K, cam2enu, resolution, conv1_w, gn_gamma, gn_beta, conv2_w, conv2_b, log_temp):
    raise NotImplementedError("write your pallas kernel here")


# speedup vs baseline: 1.5842x; 1.5842x over previous
import functools

import numpy as np
import jax
import jax.numpy as jnp
from jax import lax
from jax.experimental import pallas as pl
from jax.experimental.pallas import tpu as pltpu
from jax.experimental.pallas import tpu_sc as plsc

_INTERPRET = False

FEAT = 128
CMID = 64
NXY = 256
HF, WF = 56, 96
H, W = 224, 384
GROUPS = 16
B = 4
N = B * H * W
G = B * NXY * NXY
HB = 16
WPAD = 512


def _resize_mat(out_n, in_n):
    scale = out_n / in_n
    sample = (np.arange(out_n) + 0.5) / scale - 0.5
    d = np.abs(sample[:, None] - np.arange(in_n)[None, :])
    w = np.maximum(0.0, 1.0 - d)
    w = w / w.sum(axis=1, keepdims=True)
    return w.astype(np.float32)

_UH = jnp.asarray(_resize_mat(H, HF))
_UWT = jnp.asarray(_resize_mat(W, WF).T)


def _upw_body(x_ref, uwt_ref, o_ref):
    o_ref[...] = jnp.dot(x_ref[...], uwt_ref[...],
                         preferred_element_type=jnp.float32)


def _upsample_w(x):
    m = B * FEAT * HF
    xm = x.reshape(m, WF)
    blk = 2048
    return pl.pallas_call(
        _upw_body,
        grid=(m // blk,),
        in_specs=[pl.BlockSpec((blk, WF), lambda i: (i, 0)),
                  pl.BlockSpec((WF, W), lambda i: (0, 0))],
        out_specs=pl.BlockSpec((blk, W), lambda i: (i, 0)),
        out_shape=jax.ShapeDtypeStruct((m, W), jnp.float32),
        interpret=_INTERPRET,
    )(xm, _UWT)


CB = 16

def _uph_body(uh_ref, a_ref, o_ref):
    for j in range(CB):
        o_ref[j] = jnp.dot(uh_ref[...], a_ref[j],
                           preferred_element_type=jnp.float32)


def _upsample_h(a):
    bc = B * FEAT
    a3 = a.reshape(bc, HF, W)
    out = pl.pallas_call(
        _uph_body,
        grid=(bc // CB,),
        in_specs=[pl.BlockSpec((H, HF), lambda i: (0, 0)),
                  pl.BlockSpec((CB, HF, W), lambda i: (i, 0, 0))],
        out_specs=pl.BlockSpec((CB, H, W), lambda i: (i, 0, 0)),
        out_shape=jax.ShapeDtypeStruct((bc, H, W), jnp.float32),
        interpret=_INTERPRET,
    )(_UH, a3)
    return out.reshape(B, FEAT, H, W)


NH = H // HB

RB = HB + 16

def _tap_sum(y4, base):
    acc = jnp.zeros((CMID, HB, W), jnp.float32)
    for tap in range(9):
        ky, kx = tap // 3, tap % 3
        rstart = base + ky
        s0, s1 = max(rstart, 0), min(rstart + HB, RB)
        d0, d1 = s0 - rstart, s1 - rstart
        cstart = kx - 1
        c0 = max(cstart, 0)
        e0 = c0 - cstart
        piece = y4[tap, :, s0:s1, c0:c0 + W - e0]
        if e0:
            piece = jnp.concatenate(
                [jnp.zeros((CMID, s1 - s0, e0), jnp.float32), piece], axis=2)
        if d0:
            piece = jnp.concatenate(
                [jnp.zeros((CMID, d0, W), jnp.float32), piece], axis=1)
        if d1 < HB:
            piece = jnp.concatenate(
                [piece, jnp.zeros((CMID, HB - d1, W), jnp.float32)], axis=1)
        acc = acc + piece
    return acc


def _conv_body(xup_ref, nd_ref, wcat_ref, h_ref, p_ref, xs, sem):
    b = pl.program_id(0)
    hb = pl.program_id(1)
    r0 = hb * HB
    rs = pl.multiple_of(jnp.clip(r0 - 8, 0, H - RB), 8)

    @pl.when(jnp.logical_and(b == 0, hb == 0))
    def _init():
        xs[...] = jnp.zeros_like(xs)

    pltpu.async_copy(
        xup_ref.at[b, :, pl.ds(rs, RB), :], xs.at[0:FEAT, :, 0:W], sem).wait()
    pltpu.async_copy(
        nd_ref.at[b, pl.ds(rs, RB), :], xs.at[FEAT, :, 0:W], sem).wait()

    xflat = xs[...].reshape(FEAT + 1, RB * WPAD)
    y = jnp.dot(wcat_ref[...], xflat, preferred_element_type=jnp.float32)
    y4 = y.reshape(9, CMID, RB, WPAD)

    def _emit(base):
        acc = _tap_sum(y4, base)
        h_ref[0] = acc
        s1 = jnp.sum(acc, axis=(1, 2))
        s2 = jnp.sum(acc * acc, axis=(1, 2))
        p_ref[0, 0] = jnp.stack([s1, s2])

    @pl.when(hb == 0)
    def _top():
        _emit(-1)

    @pl.when(hb == NH - 1)
    def _bot():
        _emit((NH - 1) * HB - (H - RB) - 1)

    @pl.when(jnp.logical_and(hb != 0, hb != NH - 1))
    def _mid():
        _emit(7)


def _conv_gn_partials(xup, nd, conv1_w):
    wcat = conv1_w.transpose(2, 3, 0, 1).reshape(9 * CMID, FEAT + 1)
    return pl.pallas_call(
        _conv_body,
        grid=(B, NH),
        in_specs=[
            pl.BlockSpec(memory_space=pltpu.HBM),
            pl.BlockSpec(memory_space=pltpu.HBM),
            pl.BlockSpec((9 * CMID, FEAT + 1), lambda b, i: (0, 0)),
        ],
        out_specs=[
            pl.BlockSpec((1, CMID, HB, W), lambda b, i: (b, 0, i, 0)),
            pl.BlockSpec((1, 1, 2, CMID), lambda b, i: (b, i, 0, 0)),
        ],
        out_shape=[
            jax.ShapeDtypeStruct((B, CMID, H, W), jnp.float32),
            jax.ShapeDtypeStruct((B, NH, 2, CMID), jnp.float32),
        ],
        scratch_shapes=[
            pltpu.VMEM((FEAT + 1, RB, WPAD), jnp.float32),
            pltpu.SemaphoreType.DMA,
        ],
        interpret=_INTERPRET,
    )(xup, nd, wcat)


def _logw_body(h_ref, sc_ref, bi_ref, w2_ref, vm_ref, lw_ref, mx_ref):
    b = pl.program_id(0)
    hv = h_ref[0]
    hn = hv * sc_ref[b][:, None, None] + bi_ref[b][:, None, None]
    sil = hn / (1.0 + jnp.exp(-hn))
    lw = jnp.sum(sil * w2_ref[...][:, None, None], axis=0)
    lw_ref[0] = lw
    masked = jnp.where(vm_ref[0] > 0.0, lw, -1e30)
    mx_ref[...] = jnp.max(masked).reshape(1, 1, 1, 1)


def _logw(h, scale, bias, w2eff, validf):
    return pl.pallas_call(
        _logw_body,
        grid=(B, NH),
        in_specs=[
            pl.BlockSpec((1, CMID, HB, W), lambda b, i: (b, 0, i, 0)),
            pl.BlockSpec((B, CMID), lambda b, i: (0, 0)),
            pl.BlockSpec((B, CMID), lambda b, i: (0, 0)),
            pl.BlockSpec((CMID,), lambda b, i: (0,)),
            pl.BlockSpec((1, HB, W), lambda b, i: (b, i, 0)),
        ],
        out_specs=[
            pl.BlockSpec((1, HB, W), lambda b, i: (b, i, 0)),
            pl.BlockSpec((1, 1, 1, 1), lambda b, i: (b, i, 0, 0)),
        ],
        out_shape=[
            jax.ShapeDtypeStruct((B, H, W), jnp.float32),
            jax.ShapeDtypeStruct((B, NH, 1, 1), jnp.float32),
        ],
        interpret=_INTERPRET,
    )(h, scale, bias, w2eff, validf)


def _geometry_flipped(depth, K, cam2enu, resolution):
    nx = ny = NXY
    res = resolution.reshape(B, 1).astype(jnp.float32)
    us, vs = jnp.meshgrid(jnp.arange(W, dtype=jnp.float32),
                          jnp.arange(H, dtype=jnp.float32), indexing='xy')
    us = jnp.broadcast_to(us[None], (B, H, W))
    vs = jnp.broadcast_to(vs[None], (B, H, W))
    xs = (us - K[:, 0, 2].reshape(B, 1, 1)) * depth / K[:, 0, 0].reshape(B, 1, 1)
    ys = (vs - K[:, 1, 2].reshape(B, 1, 1)) * depth / K[:, 1, 1].reshape(B, 1, 1)
    pts_cam = jnp.stack([xs, ys, depth], axis=-1).reshape(B, -1, 3)
    pts_enu = (pts_cam @ jnp.swapaxes(cam2enu[:, :3, :3], -1, -2)
               + cam2enu[:, :3, 3][:, None, :])
    y_min = -ny * res / 2.0
    vx = jnp.floor(pts_enu[..., 0] / res).astype(jnp.int32)
    vy = jnp.floor((pts_enu[..., 1] - y_min) / res).astype(jnp.int32)
    valid = (vx >= 0) & (vx < nx) & (vy >= 0) & (vy < ny)
    vx = vx.reshape(-1)
    vy = vy.reshape(-1)
    valid = valid.reshape(-1)
    boff = (jnp.arange(B, dtype=jnp.int32) * (nx * ny))[:, None]
    boff = jnp.broadcast_to(boff, (B, H * W)).reshape(-1)
    gflip = (nx - 1 - vx) * ny + (ny - 1 - vy) + boff
    spread = jnp.arange(N, dtype=jnp.int32) & (G - 1)
    idx = jnp.where(valid, gflip, spread)
    return valid.astype(jnp.float32), idx


def kernel(x, depth, K, cam2enu, resolution, conv1_w, gn_gamma, gn_beta,
           conv2_w, conv2_b, log_temp):
    validf, idx = _geometry_flipped(depth, K, cam2enu, resolution)
    clean = jnp.nan_to_num(depth, nan=0.0, posinf=100.0, neginf=0.0)
    nd = jnp.clip(clean, 0.0, 100.0) / 100.0

    a = _upsample_w(x)
    xup = _upsample_h(a)

    h, parts = _conv_gn_partials(xup, nd, conv1_w)
    s = parts.sum(axis=1)
    cnt = 4.0 * H * W
    sg = s.reshape(B, 2, GROUPS, CMID // GROUPS).sum(axis=3)
    mu = sg[:, 0] / cnt
    var = sg[:, 1] / cnt - mu * mu
    inv = 1.0 / jnp.sqrt(var + 1e-5)
    mu_c = jnp.repeat(mu, CMID // GROUPS, axis=1)
    inv_c = jnp.repeat(inv, CMID // GROUPS, axis=1)
    scale = inv_c * gn_gamma[None, :]
    bias = gn_beta[None, :] - mu_c * scale

    inv_temp = 1.0 / jnp.exp(log_temp)
    w2eff = conv2_w[:, :, 0, 0].reshape(CMID) * inv_temp
    validm = validf.reshape(B, H, W)
    lw, bmax = _logw(h, scale, bias, w2eff, validm)
    lw = lw + conv2_b[0] * inv_temp
    lwmax = jnp.max(bmax) + conv2_b[0] * inv_temp

    lwf = lw.reshape(N)
    wgt = validf * jnp.exp(lwf - lwmax)
    ws = jnp.zeros((G,), jnp.float32).at[idx].add(wgt)
    feat = xup.transpose(0, 2, 3, 1).reshape(N, FEAT)
    raw = jnp.zeros((G, FEAT), jnp.float32).at[idx].add(wgt[:, None] * feat)
    bev = raw / jnp.maximum(ws, 1e-4)[:, None]
    bev_emb = bev.reshape(B, NXY, NXY, FEAT).transpose(0, 3, 1, 2)
    bev_mask = (ws > 1e-6).astype(x.dtype).reshape(B, 1, NXY, NXY)
    return bev_emb, bev_mask
